# baseline (speedup 1.0000x reference)
import functools

import jax
import jax.numpy as jnp
from jax import lax
from jax.experimental import pallas as pl
from jax.experimental.pallas import tpu as pltpu
from jax.experimental.pallas import tpu_sc as plsc

EMB_DIM = 32
CHUNK = 1600


@functools.lru_cache(maxsize=None)
def _make_gather(num_rows: int):
    info = plsc.get_sparse_core_info()
    nc, ns = info.num_cores, info.num_subcores
    nw = nc * ns
    n_chunks = num_rows // CHUNK
    k_per_w = n_chunks // nw
    assert n_chunks * CHUNK == num_rows and k_per_w * nw == n_chunks

    mesh = plsc.VectorSubcoreMesh(core_axis_name="c", subcore_axis_name="s")

    @functools.partial(
        pl.kernel,
        mesh=mesh,
        out_type=jax.ShapeDtypeStruct((num_rows, EMB_DIM), jnp.float32),
        compiler_params=pltpu.CompilerParams(use_tc_tiling_on_sc=False),
        scratch_types=[
            pltpu.VMEM((k_per_w * CHUNK,), jnp.int32),
            pltpu.VMEM((2, CHUNK, EMB_DIM), jnp.float32),
            pltpu.SemaphoreType.DMA,
            pltpu.SemaphoreType.DMA,
            pltpu.SemaphoreType.DMA,
            pltpu.SemaphoreType.DMA,
        ],
    )
    def gather_kernel(idx_hbm, table_hbm, out_hbm, idx_v, bufs, g0, g1, w0, w1):
        wid = lax.axis_index("s") * nc + lax.axis_index("c")
        base = wid * (k_per_w * CHUNK)
        pltpu.sync_copy(idx_hbm.at[pl.ds(base, k_per_w * CHUNK)], idx_v)
        sem_g = (g0, g1)
        sem_w = (w0, w1)

        gathers = [None, None]
        writes = [None, None]
        for j in range(k_per_w):
            b = j % 2
            if writes[b] is not None:
                writes[b].wait()
            idx_c = idx_v.at[pl.ds(j * CHUNK, CHUNK)]
            gathers[b] = pltpu.async_copy(
                table_hbm.at[idx_c], bufs.at[b], sem_g[b])
            pb = (j - 1) % 2
            if j > 0:
                gathers[pb].wait()
                writes[pb] = pltpu.async_copy(
                    bufs.at[pb],
                    out_hbm.at[pl.ds(base + (j - 1) * CHUNK, CHUNK)],
                    sem_w[pb])
        lb = (k_per_w - 1) % 2
        gathers[lb].wait()
        writes[lb] = pltpu.async_copy(
            bufs.at[lb],
            out_hbm.at[pl.ds(base + (k_per_w - 1) * CHUNK, CHUNK)],
            sem_w[lb])
        writes[lb].wait()
        if k_per_w > 1:
            writes[1 - lb].wait()

    return gather_kernel


def kernel(indices, table):
    b, h = indices.shape
    num_rows = b * h
    idx_flat = indices.reshape(num_rows).astype(jnp.int32)
    out = _make_gather(num_rows)(idx_flat, table)
    return out.reshape(b, h, EMB_DIM)


# speedup vs baseline: 1.0923x; 1.0923x over previous
import functools

import jax
import jax.numpy as jnp
from jax import lax
from jax.experimental import pallas as pl
from jax.experimental.pallas import tpu as pltpu
from jax.experimental.pallas import tpu_sc as plsc

EMB_DIM = 32
LANE = 16
BBLK = 128


@functools.lru_cache(maxsize=None)
def _make_gather(batch: int, hist: int):
    info = plsc.get_sparse_core_info()
    nc, ns = info.num_cores, info.num_subcores
    nw = nc * ns
    nbb = batch // BBLK
    assert nbb == nw and batch % BBLK == 0
    neb = EMB_DIM // 8

    mesh = plsc.VectorSubcoreMesh(core_axis_name="c", subcore_axis_name="s")

    @functools.partial(
        pl.kernel,
        mesh=mesh,
        out_type=jax.ShapeDtypeStruct((hist * neb * nbb * 8, BBLK), jnp.float32),
        compiler_params=pltpu.CompilerParams(
            use_tc_tiling_on_sc=False, needs_layout_passes=False),
        scratch_types=[
            pltpu.VMEM((hist, BBLK), jnp.int32),
            pltpu.VMEM((BBLK, EMB_DIM), jnp.float32),
            pltpu.VMEM((neb, 8, BBLK), jnp.float32),
            pltpu.SemaphoreType.DMA,
        ],
    )
    def gather_kernel(idx_hbm, table_hbm, out_hbm, idx_v, rows_v, tile_v, sem):
        w = lax.axis_index("s") * nc + lax.axis_index("c")
        pltpu.sync_copy(idx_hbm.at[:, pl.ds(w * BBLK, BBLK)], idx_v)
        lane = lax.iota(jnp.int32, LANE)

        def body(h, carry):
            pltpu.async_copy(table_hbm.at[idx_v.at[h]], rows_v, sem).wait()
            for e in range(EMB_DIM):
                col = jnp.full((LANE,), e, jnp.int32)
                for i0 in range(BBLK // LANE):
                    v = plsc.load_gather(rows_v, [lane + i0 * LANE, col])
                    tile_v[e // 8, e % 8, pl.ds(i0 * LANE, LANE)] = v
            row0 = ((h * neb) * nbb + w) * 8
            for eb in range(neb):
                pltpu.sync_copy(
                    tile_v.at[eb],
                    out_hbm.at[pl.ds(row0 + eb * nbb * 8, 8)])
            return carry

        lax.fori_loop(0, hist, body, 0)

    return gather_kernel


def kernel(indices, table):
    b, h = indices.shape
    idx_t = indices.T.astype(jnp.int32)
    out2d = _make_gather(b, h)(idx_t, table)
    out5 = out2d.reshape(h, EMB_DIM // 8, b // BBLK, 8, BBLK)
    return out5.transpose(2, 4, 0, 1, 3).reshape(b, h, EMB_DIM)
